# baseline (speedup 1.0000x reference)
import jax
import jax.numpy as jnp
from jax import lax
from jax.experimental import pallas as pl
from jax.experimental.pallas import tpu as pltpu
from jax.experimental.pallas import tpu_sc as plsc

_D = 768
_MAXP = 64
_SCALE = float(_D) ** 0.5

_NC = 2
_NS = 16
_NW = _NC * _NS

_Q = 16
_NBUF = 4
_LANES = 16
_VPR = _D // _LANES


def _fma_chunk(g, pos_v, b):
    poff = b * _Q

    def row_body(r, carry):
        for u in range(_VPR):
            sl = pl.ds(u * _LANES, _LANES)
            g[b, r, sl] = g[b, r, sl] * _SCALE + pos_v[poff + r, sl]
        return carry

    lax.fori_loop(0, _Q, row_body, 0, unroll=False)


def _body(ids_hbm, table_hbm, pos_hbm, out_hbm, idx_v, pos_v, g,
          sem_g, sem_s):
    wid = lax.axis_index("s") * _NC + lax.axis_index("c")
    seq_per_w = ids_hbm.shape[0] // _NW
    wseq = wid * seq_per_w

    pltpu.sync_copy(ids_hbm.at[pl.ds(wseq, seq_per_w)], idx_v)
    pltpu.sync_copy(pos_hbm, pos_v)

    def start_gather(s, b):
        pltpu.async_copy(table_hbm.at[idx_v.at[s, pl.ds(b * _Q, _Q)]],
                         g.at[b], sem_g[b])

    def start_scatter(s, b):
        pltpu.async_copy(g.at[b], out_hbm.at[wseq + s, pl.ds(b * _Q, _Q)],
                         sem_s[b])

    def wait_gather(b):
        pltpu.make_async_copy(table_hbm.at[idx_v.at[0, pl.ds(0, _Q)]],
                              g.at[b], sem_g[b]).wait()

    def wait_scatter(b):
        pltpu.make_async_copy(g.at[b], out_hbm.at[0, pl.ds(0, _Q)],
                              sem_s[b]).wait()

    for b in range(_NBUF - 1):
        start_gather(0, b)

    def process(s, b, first, last):
        wait_gather(b)
        _fma_chunk(g, pos_v, b)
        start_scatter(s, b)
        if not last:
            bn = (b + _NBUF - 1) % _NBUF
            dg = (b + _NBUF - 1) // _NBUF
            if not first:
                wait_scatter(bn)
            start_gather(s + dg, bn)

    for b in range(_NBUF):
        process(0, b, first=(b == 0), last=False)

    def seq_body(s, carry):
        for b in range(_NBUF):
            process(s, b, first=False, last=False)
        return carry

    lax.fori_loop(1, seq_per_w - 1, seq_body, 0, unroll=False)

    for b in range(_NBUF):
        process(seq_per_w - 1, b, first=False, last=(b > 0))

    for b in range(_NBUF):
        wait_scatter(b)


@jax.jit
def kernel(input_ids, token_embedding, position_embedding):
    batch, seq = input_ids.shape
    seq_per_w = batch // _NW

    mesh = plsc.VectorSubcoreMesh(core_axis_name="c", subcore_axis_name="s")
    run = pl.kernel(
        _body,
        out_type=jax.ShapeDtypeStruct((batch, seq, _D), jnp.float32),
        mesh=mesh,
        compiler_params=pltpu.CompilerParams(use_tc_tiling_on_sc=False),
        scratch_types=[
            pltpu.VMEM((seq_per_w, _MAXP), jnp.int32),
            pltpu.VMEM((_MAXP, _D), jnp.float32),
            pltpu.VMEM((_NBUF, _Q, _D), jnp.float32),
            [pltpu.SemaphoreType.DMA] * _NBUF,
            [pltpu.SemaphoreType.DMA] * _NBUF,
        ],
    )
    return run(input_ids.astype(jnp.int32), token_embedding,
               position_embedding)
